# baseline (speedup 1.0000x reference)
import functools

import jax
import jax.numpy as jnp
from jax import lax
from jax.experimental import pallas as pl
from jax.experimental.pallas import tpu as pltpu
from jax.experimental.pallas import tpu_sc as plsc

EMBED = 1024
VOCAB = 256
PAD = 128
NC = 2
NS = 16
NW = NC * NS
LANES = 16

NCG = 8
NSQ = NW // NCG
COLS = EMBED // NCG
T = 64
NBUF = 2


def _body(tbl_hbm, idx_hbm, pos_hbm, out_hbm,
          tbl_v, idx_v, pos_v, out_v,
          in_sem0, in_sem1, out_sem0, out_sem1, B, S):
    cid = lax.axis_index("c")
    sid = lax.axis_index("s")
    wid = sid * NC + cid
    cg = wid % NCG
    sq = wid // NCG
    c0 = cg * COLS
    s_per = S // NSQ
    sq0 = sq * s_per
    n_ch = s_per // T

    in_sems = (in_sem0, in_sem1)
    out_sems = (out_sem0, out_sem1)

    pltpu.sync_copy(tbl_hbm.at[:, pl.ds(c0, COLS)], tbl_v)
    for j in range(COLS // LANES):
        tbl_v[PAD, pl.ds(j * LANES, LANES)] = jnp.zeros((LANES,), jnp.float32)
    pltpu.sync_copy(idx_hbm.at[:, pl.ds(sq0, s_per)], idx_v)

    def in_copies(ci, slot):
        s0 = sq0 + ci * T
        return (
            pltpu.make_async_copy(
                pos_hbm.at[pl.ds(s0, T), pl.ds(c0, COLS)],
                pos_v.at[slot], in_sems[slot]),
        )

    def out_copies(ci, slot):
        s0 = sq0 + ci * T
        return tuple(
            pltpu.make_async_copy(
                out_v.at[slot, b],
                out_hbm.at[b, pl.ds(s0, T), pl.ds(c0, COLS)],
                out_sems[slot])
            for b in range(B))

    def compute(ci, slot):
        pv = pos_v.at[slot]
        ov = out_v.at[slot]

        def t16_body(t16, _):
            base = t16 * LANES
            idxvecs = [idx_v[b, pl.ds(ci * T + base, LANES)] for b in range(B)]
            for lane in range(LANES):
                sl = base + lane
                idxs = [idxvecs[b][lane] for b in range(B)]
                for j in range(COLS // LANES):
                    csl = pl.ds(j * LANES, LANES)
                    posvec = pv[sl, csl]
                    for b in range(B):
                        ov[b, sl, csl] = tbl_v[idxs[b], csl] + posvec
            return 0
        lax.fori_loop(0, T // LANES, t16_body, 0)

    for d in in_copies(0, 0):
        d.start()

    def outer(ci2, _):
        for sub in range(NBUF):
            ci = ci2 * NBUF + sub
            for d in in_copies(ci, sub):
                d.wait()

            @pl.when(ci + 1 < n_ch)
            def _prefetch():
                for d in in_copies(ci + 1, 1 - sub):
                    d.start()

            @pl.when(ci2 > 0)
            def _reuse_guard():
                for d in out_copies(ci - NBUF, sub):
                    d.wait()

            compute(ci, sub)
            for d in out_copies(ci, sub):
                d.start()
        return 0
    lax.fori_loop(0, n_ch // NBUF, outer, 0)

    for d in out_copies(n_ch - 2, 0):
        d.wait()
    for d in out_copies(n_ch - 1, 1):
        d.wait()


def kernel(inputs, value_table, pos_table):
    B, S = inputs.shape

    mesh = plsc.VectorSubcoreMesh(
        core_axis_name="c", subcore_axis_name="s",
        num_cores=NC, num_subcores=NS)

    k = functools.partial(
        pl.kernel,
        out_type=jax.ShapeDtypeStruct((B, S, EMBED), jnp.float32),
        mesh=mesh,
        scratch_types=[
            pltpu.VMEM((VOCAB, COLS), jnp.float32),
            pltpu.VMEM((B, S // NSQ), jnp.int32),
            pltpu.VMEM((NBUF, T, COLS), jnp.float32),
            pltpu.VMEM((NBUF, B, T, COLS), jnp.float32),
            pltpu.SemaphoreType.DMA,
            pltpu.SemaphoreType.DMA,
            pltpu.SemaphoreType.DMA,
            pltpu.SemaphoreType.DMA,
        ],
    )(functools.partial(_body, B=B, S=S))

    return k(value_table, inputs, pos_table)


# speedup vs baseline: 3.2879x; 3.2879x over previous
import functools

import jax
import jax.numpy as jnp
from jax import lax
from jax.experimental import pallas as pl
from jax.experimental.pallas import tpu as pltpu
from jax.experimental.pallas import tpu_sc as plsc

EMBED = 1024
VOCAB = 256
PAD = 128
ZROW = VOCAB
NC = 2
NS = 16
NW = NC * NS
LANES = 16

CH = 8
NBUF = 2


def _body(tbl_hbm, idx_hbm, pos_hbm, out_hbm,
          idx_v, pos_v, rows_v,
          in_sem0, in_sem1, out_sem0, out_sem1, B, S):
    cid = lax.axis_index("c")
    sid = lax.axis_index("s")
    wid = sid * NC + cid

    n_per_w = S // NW
    n_ch = n_per_w // CH
    s_base = wid * n_per_w

    in_sems = (in_sem0, in_sem1)
    out_sems = (out_sem0, out_sem1)

    pltpu.sync_copy(idx_hbm.at[:, pl.ds(s_base, n_per_w)], idx_v)
    for b in range(B):
        for j in range(n_per_w // LANES):
            sl = pl.ds(j * LANES, LANES)
            v = idx_v[b, sl]
            idx_v[b, sl] = jnp.where(v == PAD, ZROW, v)

    def in_copies(ci, slot):
        s0 = s_base + ci * CH
        cps = [pltpu.make_async_copy(
            pos_hbm.at[pl.ds(s0, CH)], pos_v.at[slot], in_sems[slot])]
        for b in range(B):
            cps.append(pltpu.make_async_copy(
                tbl_hbm.at[idx_v.at[b, pl.ds(ci * CH, CH)]],
                rows_v.at[slot, b], in_sems[slot]))
        return cps

    def out_copies(ci, slot):
        s0 = s_base + ci * CH
        return tuple(
            pltpu.make_async_copy(
                rows_v.at[slot, b],
                out_hbm.at[pl.ds(b * S + s0, CH)], out_sems[slot])
            for b in range(B))

    def compute(slot):
        def r_body(r, _):
            for j in range(EMBED // LANES):
                csl = pl.ds(j * LANES, LANES)
                posvec = pos_v[slot, r, csl]
                for b in range(B):
                    rows_v[slot, b, r, csl] = rows_v[slot, b, r, csl] + posvec
            return 0
        lax.fori_loop(0, CH, r_body, 0)

    for d in in_copies(0, 0):
        d.start()

    def outer(ci2, _):
        for sub in range(NBUF):
            ci = ci2 * NBUF + sub
            for d in in_copies(ci, sub):
                d.wait()

            @pl.when(jnp.logical_and(ci + 1 < n_ch, ci >= 1))
            def _retire_other():
                for d in out_copies(ci - 1, 1 - sub):
                    d.wait()

            @pl.when(ci + 1 < n_ch)
            def _prefetch():
                for d in in_copies(ci + 1, 1 - sub):
                    d.start()

            compute(sub)
            for d in out_copies(ci, sub):
                d.start()
        return 0
    lax.fori_loop(0, n_ch // NBUF, outer, 0)

    for d in out_copies(n_ch - 2, 0):
        d.wait()
    for d in out_copies(n_ch - 1, 1):
        d.wait()


def kernel(inputs, value_table, pos_table):
    B, S = inputs.shape
    tbl_pad = jnp.concatenate(
        [value_table, jnp.zeros((8, EMBED), jnp.float32)], axis=0)

    mesh = plsc.VectorSubcoreMesh(
        core_axis_name="c", subcore_axis_name="s",
        num_cores=NC, num_subcores=NS)

    k = functools.partial(
        pl.kernel,
        out_type=jax.ShapeDtypeStruct((B * S, EMBED), jnp.float32),
        mesh=mesh,
        scratch_types=[
            pltpu.VMEM((B, S // NW), jnp.int32),
            pltpu.VMEM((NBUF, CH, EMBED), jnp.float32),
            pltpu.VMEM((NBUF, B, CH, EMBED), jnp.float32),
            pltpu.SemaphoreType.DMA,
            pltpu.SemaphoreType.DMA,
            pltpu.SemaphoreType.DMA,
            pltpu.SemaphoreType.DMA,
        ],
    )(functools.partial(_body, B=B, S=S))

    out = k(tbl_pad, inputs, pos_table)
    return out.reshape(B, S, EMBED)
